# Initial kernel scaffold; baseline (speedup 1.0000x reference)
import jax
import jax.numpy as jnp
from jax.experimental import pallas as pl


def kernel(inputs, E_weight):
    raise NotImplementedError("write your pallas kernel here")


# speedup vs baseline: 1.8879x; 1.8879x over previous
import functools

import jax
import jax.numpy as jnp
from jax import lax
from jax.experimental import pallas as pl
from jax.experimental.pallas import tpu as pltpu
from jax.experimental.pallas import tpu_sc as plsc

K = 1024
D = 64
BETA = 0.25
N_BATCH = 16
T = 1024
N_TOK = N_BATCH * T


def _dist_kernel(x_ref, e_ref, idx_ref, loss_ref):
    ze = x_ref[0]
    e = e_ref[...]
    en = jnp.sum(e * e, axis=1, keepdims=True)
    zn = jnp.sum(ze * ze, axis=0, keepdims=True)
    s = lax.dot_general(e, ze, (((1,), (0,)), ((), ())),
                        preferred_element_type=jnp.float32)
    dist = (zn + en) - 2.0 * s
    minv = jnp.min(dist, axis=0, keepdims=True)
    kio = lax.broadcasted_iota(jnp.int32, (K, T), 0)
    idx = jnp.min(jnp.where(dist == minv, kio, K), axis=0, keepdims=True)
    idx_ref[0] = idx
    part = jnp.sum(minv, keepdims=True)

    @pl.when(pl.program_id(0) == 0)
    def _():
        loss_ref[...] = part

    @pl.when(pl.program_id(0) != 0)
    def _():
        loss_ref[...] += part


def _distances(x, e_weight):
    return pl.pallas_call(
        _dist_kernel,
        grid=(N_BATCH,),
        in_specs=[
            pl.BlockSpec((1, D, T), lambda n: (n, 0, 0)),
            pl.BlockSpec((K, D), lambda n: (0, 0)),
        ],
        out_specs=[
            pl.BlockSpec((1, 1, T), lambda n: (n, 0, 0)),
            pl.BlockSpec((1, 1), lambda n: (0, 0)),
        ],
        out_shape=[
            jax.ShapeDtypeStruct((N_BATCH, 1, T), jnp.int32),
            jax.ShapeDtypeStruct((1, 1), jnp.float32),
        ],
    )(x, e_weight)


_NC, _NS = 2, 16
_NW = _NC * _NS
_TPW = N_TOK // _NW
_CHUNK = 128
_NCHUNK = _TPW // _CHUNK


def _sc_gather_kernel(idx_hbm, table_hbm, out_hbm, hist_hbm,
                      idx_v, rows_v, hist_v, sem):
    wid = lax.axis_index("s") * _NC + lax.axis_index("c")
    base = wid * _NCHUNK
    pltpu.sync_copy(idx_hbm.at[pl.ds(base, _NCHUNK)], idx_v)
    handles = [
        pltpu.async_copy(table_hbm.at[idx_v.at[j]], rows_v.at[j], sem)
        for j in range(_NCHUNK)
    ]
    zeros = jnp.zeros((16,), jnp.float32)
    for i in range(K // 16):
        hist_v[pl.ds(i * 16, 16)] = zeros
    ones = jnp.ones((16,), jnp.float32)
    for j in range(_NCHUNK):
        for t in range(_CHUNK // 16):
            v = idx_v[j, pl.ds(t * 16, 16)]
            plsc.addupdate_scatter(hist_v, [v], ones)
    for h in handles:
        h.wait()
    pltpu.sync_copy(rows_v, out_hbm.at[pl.ds(base, _NCHUNK)])
    pltpu.sync_copy(hist_v, hist_hbm.at[wid])


@functools.cache
def _sc_gather():
    return pl.kernel(
        _sc_gather_kernel,
        mesh=plsc.VectorSubcoreMesh(core_axis_name="c", subcore_axis_name="s"),
        out_type=[
            jax.ShapeDtypeStruct((_NW * _NCHUNK, _CHUNK, D), jnp.float32),
            jax.ShapeDtypeStruct((_NW, K), jnp.float32),
        ],
        scratch_types=[
            pltpu.VMEM((_NCHUNK, _CHUNK), jnp.int32),
            pltpu.VMEM((_NCHUNK, _CHUNK, D), jnp.float32),
            pltpu.VMEM((K,), jnp.float32),
            pltpu.SemaphoreType.DMA,
        ],
        compiler_params=pltpu.CompilerParams(
            needs_layout_passes=False, use_tc_tiling_on_sc=False),
    )


def _finalize_kernel(hist_ref, loss_ref, eq_ref, el_ref, ql_ref, est_ref):
    h = hist_ref[...]
    counts = jnp.sum(h, axis=0, keepdims=True)
    tot = jnp.sum(counts)
    prob = counts / tot
    log_prob = jnp.log2(prob + 1e-10)
    ent = -jnp.sum(prob * log_prob, keepdims=True)
    est_ref[...] = jnp.exp(ent * 0.6931471805599453)
    el = loss_ref[...] * (1.0 / (N_TOK * D))
    el_ref[...] = el
    ql_ref[...] = el
    eq_ref[...] = el + BETA * el


def _finalize(hist, loss_sum):
    return pl.pallas_call(
        _finalize_kernel,
        in_specs=[
            pl.BlockSpec((_NW, K), lambda: (0, 0)),
            pl.BlockSpec((1, 1), lambda: (0, 0)),
        ],
        out_specs=[pl.BlockSpec((1, 1), lambda: (0, 0))] * 4,
        out_shape=[jax.ShapeDtypeStruct((1, 1), jnp.float32)] * 4,
    )(hist, loss_sum)


def kernel(inputs, E_weight):
    x = inputs.reshape(N_BATCH, D, T)
    idx3, loss_sum = _distances(x, E_weight)
    idx128 = idx3.reshape(_NW * _NCHUNK, _CHUNK)
    zq_rows, hist = _sc_gather()(idx128, E_weight)
    eq, el, ql, est = _finalize(hist, loss_sum)
    zq = zq_rows.reshape(N_BATCH, 32, 32, D).transpose(0, 3, 1, 2)
    return (eq[0, 0], zq, el[0, 0], ql[0, 0], est[0, 0])
